# baseline (speedup 1.0000x reference)
import jax
import jax.numpy as jnp
from jax import lax
from jax.experimental import pallas as pl
from jax.experimental.pallas import tpu as pltpu
from jax.experimental.pallas import tpu_sc as plsc

NC = 2
NS = 16
NW = NC * NS

BATCH = 16384
EMBED = 64
PITCH = 2 * EMBED
CROWS = 1000
QROWS = 1000000
BPW = BATCH // NW
CHUNK = 128
NCH = BPW // CHUNK
LANES = 16
KV = EMBED // LANES

RBLK = 4000


def _repack_kernel(rows_blk):
  def body(top_ref, bot_ref, o_ref):
    o_ref[...] = jnp.concatenate([top_ref[...], bot_ref[...]], axis=1)

  def repack(table):
    half = table.shape[0] // 2
    return pl.pallas_call(
        body,
        grid=(half // rows_blk,),
        in_specs=[
            pl.BlockSpec((rows_blk, EMBED), lambda i: (i, 0)),
            pl.BlockSpec((rows_blk, EMBED), lambda i: (i, 0)),
        ],
        out_specs=pl.BlockSpec((rows_blk, PITCH), lambda i: (i, 0)),
        out_shape=jax.ShapeDtypeStruct((half, PITCH), jnp.float32),
    )(table[:half], table[half:])

  return repack


def _make_sc_kernel():
  mesh = plsc.VectorSubcoreMesh(core_axis_name="c", subcore_axis_name="s")

  @pl.kernel(
      out_type=jax.ShapeDtypeStruct((BATCH, PITCH), jnp.float32),
      mesh=mesh,
      scratch_types=[
          pltpu.VMEM((2 * NCH, CHUNK), jnp.int32),
          pltpu.VMEM((2 * NCH, CHUNK), jnp.int32),
          pltpu.VMEM((2, CHUNK, PITCH), jnp.float32),
          pltpu.VMEM((2, CHUNK, PITCH), jnp.float32),
          pltpu.VMEM((2, CHUNK, PITCH), jnp.float32),
          pltpu.SemaphoreType.DMA,
          pltpu.SemaphoreType.DMA,
      ],
  )
  def k(hbit_hbm, hidx_hbm, cpk_hbm, qpk_hbm, out_hbm,
        hbit_v, hidx_v, bufc, bufq, mix, gsem, osem):
    wid = lax.axis_index("s") * NC + lax.axis_index("c")
    base = wid * BPW

    pltpu.sync_copy(hbit_hbm.at[wid], hbit_v)
    pltpu.sync_copy(hidx_hbm.at[wid], hidx_v)

    def issue(g, slot):
      pltpu.async_copy(cpk_hbm.at[hidx_v.at[g]], bufc.at[slot], gsem)
      pltpu.async_copy(qpk_hbm.at[hidx_v.at[NCH + g]], bufq.at[slot], gsem)

    issue(0, 0)

    def body(g, _):
      slot = g % 2

      @pl.when(g + 1 < NCH)
      def _():
        issue(g + 1, (g + 1) % 2)

      pltpu.make_async_copy(cpk_hbm.at[hidx_v.at[0]], bufc.at[slot], gsem).wait()
      pltpu.make_async_copy(cpk_hbm.at[hidx_v.at[0]], bufq.at[slot], gsem).wait()

      for rv in range(CHUNK // LANES):
        cvec = hbit_v[g, pl.ds(rv * LANES, LANES)]
        qvec = hbit_v[NCH + g, pl.ds(rv * LANES, LANES)]
        for l in range(LANES):
          r = rv * LANES + l
          ch = cvec[l]
          qh = qvec[l]
          for t in range(KV):
            mix[slot, r, pl.ds(t * LANES, LANES)] = bufc[
                slot, r, pl.ds(ch + t * LANES, LANES)
            ]
            mix[slot, r, pl.ds(EMBED + t * LANES, LANES)] = bufq[
                slot, r, pl.ds(qh + t * LANES, LANES)
            ]

      @pl.when(g >= 2)
      def _():
        pltpu.make_async_copy(
            mix.at[slot], out_hbm.at[pl.ds(base, CHUNK)], osem
        ).wait()

      pltpu.async_copy(
          mix.at[slot], out_hbm.at[pl.ds(base + g * CHUNK, CHUNK)], osem
      )
      return 0

    lax.fori_loop(0, NCH, body, 0)
    for _ in range(2):
      pltpu.make_async_copy(
          mix.at[0], out_hbm.at[pl.ds(base, CHUNK)], osem
      ).wait()

  return k


_sc_kernel = _make_sc_kernel()
_repack_q = _repack_kernel(RBLK)
_repack_c = _repack_kernel(CROWS // 2)


@jax.jit
def kernel(x, color_table, question_table):
  xc = x[:, 0].astype(jnp.int32).reshape(NW, NCH, CHUNK)
  xq = x[:, 1].astype(jnp.int32).reshape(NW, NCH, CHUNK)
  ch = CROWS // 2
  qh = QROWS // 2
  hxc = jnp.where(xc < ch, xc, xc - ch)
  hxq = jnp.where(xq < qh, xq, xq - qh)
  bxc = jnp.where(xc < ch, 0, EMBED)
  bxq = jnp.where(xq < qh, 0, EMBED)
  hidx_all = jnp.concatenate([hxc, hxq], axis=1)
  hbit_all = jnp.concatenate([bxc, bxq], axis=1)
  cpk = _repack_c(color_table)
  qpk = _repack_q(question_table)
  return _sc_kernel(hbit_all, hidx_all, cpk, qpk)


# speedup vs baseline: 1.0169x; 1.0169x over previous
import jax
import jax.numpy as jnp
from jax import lax
from jax.experimental import pallas as pl
from jax.experimental.pallas import tpu as pltpu
from jax.experimental.pallas import tpu_sc as plsc

NC = 2
NS = 16
NW = NC * NS

BATCH = 16384
EMBED = 64
PITCH = 2 * EMBED
CROWS = 1000
QROWS = 1000000
BPW = BATCH // NW
CHUNK = 128
NCH = BPW // CHUNK
LANES = 16
KV = EMBED // LANES

RBLK = 4000


def _repack_kernel(rows_blk):
  def body(top_ref, bot_ref, o_ref):
    o_ref[...] = jnp.concatenate([top_ref[...], bot_ref[...]], axis=1)

  def repack(table):
    half = table.shape[0] // 2
    return pl.pallas_call(
        body,
        grid=(half // rows_blk,),
        in_specs=[
            pl.BlockSpec((rows_blk, EMBED), lambda i: (i, 0)),
            pl.BlockSpec((rows_blk, EMBED), lambda i: (i, 0)),
        ],
        out_specs=pl.BlockSpec((rows_blk, PITCH), lambda i: (i, 0)),
        out_shape=jax.ShapeDtypeStruct((half, PITCH), jnp.float32),
    )(table[:half], table[half:])

  return repack


def _make_sc_kernel():
  mesh = plsc.VectorSubcoreMesh(core_axis_name="c", subcore_axis_name="s")

  @pl.kernel(
      out_type=jax.ShapeDtypeStruct((BATCH, PITCH), jnp.float32),
      mesh=mesh,
      scratch_types=[
          pltpu.VMEM((2 * NCH, CHUNK), jnp.int32),
          pltpu.VMEM((2 * NCH, CHUNK), jnp.int32),
          pltpu.VMEM((2, CHUNK, PITCH), jnp.float32),
          pltpu.VMEM((2, CHUNK, PITCH), jnp.float32),
          pltpu.VMEM((2, CHUNK, PITCH), jnp.float32),
          pltpu.SemaphoreType.DMA,
          pltpu.SemaphoreType.DMA,
      ],
  )
  def k(hbit_hbm, hidx_hbm, cpk_hbm, qpk_hbm, out_hbm,
        hbit_v, hidx_v, bufc, bufq, mix, gsem, osem):
    wid = lax.axis_index("s") * NC + lax.axis_index("c")
    base = wid * BPW

    pltpu.sync_copy(hbit_hbm.at[wid], hbit_v)
    pltpu.sync_copy(hidx_hbm.at[wid], hidx_v)

    def issue(g, slot):
      pltpu.async_copy(cpk_hbm.at[hidx_v.at[g]], bufc.at[slot], gsem)
      pltpu.async_copy(qpk_hbm.at[hidx_v.at[NCH + g]], bufq.at[slot], gsem)

    issue(0, 0)

    def body(g, _):
      slot = g % 2

      @pl.when(g + 1 < NCH)
      def _():
        issue(g + 1, (g + 1) % 2)

      pltpu.make_async_copy(cpk_hbm.at[hidx_v.at[0]], bufc.at[slot], gsem).wait()
      pltpu.make_async_copy(cpk_hbm.at[hidx_v.at[0]], bufq.at[slot], gsem).wait()

      for rv in range(CHUNK // LANES):
        cvec = hbit_v[g, pl.ds(rv * LANES, LANES)]
        qvec = hbit_v[NCH + g, pl.ds(rv * LANES, LANES)]
        for l in range(LANES):
          r = rv * LANES + l
          ch = cvec[l]
          qh = qvec[l]
          for t in range(KV):
            mix[slot, r, pl.ds(t * LANES, LANES)] = bufc[
                slot, r, pl.ds(ch + t * LANES, LANES)
            ]
            mix[slot, r, pl.ds(EMBED + t * LANES, LANES)] = bufq[
                slot, r, pl.ds(qh + t * LANES, LANES)
            ]

      @pl.when(g >= 2)
      def _():
        pltpu.make_async_copy(
            mix.at[slot], out_hbm.at[pl.ds(base, CHUNK)], osem
        ).wait()

      pltpu.async_copy(
          mix.at[slot], out_hbm.at[pl.ds(base + g * CHUNK, CHUNK)], osem
      )
      return 0

    lax.fori_loop(0, NCH, body, 0)
    for _ in range(2):
      pltpu.make_async_copy(
          mix.at[0], out_hbm.at[pl.ds(base, CHUNK)], osem
      ).wait()

  return k


_sc_kernel = _make_sc_kernel()
_repack_q = _repack_kernel(RBLK)
_repack_c = _repack_kernel(CROWS // 2)


@jax.jit
def kernel(x, color_table, question_table):
  xc = x[:, 0].astype(jnp.int32).reshape(NW, NCH, CHUNK)
  xq = x[:, 1].astype(jnp.int32).reshape(NW, NCH, CHUNK)
  ch = CROWS // 2
  qh = QROWS // 2
  hxc = jnp.where(xc < ch, xc, xc - ch)
  hxq = jnp.where(xq < qh, xq, xq - qh)
  bxc = jnp.where(xc < ch, 0, EMBED)
  bxq = jnp.where(xq < qh, 0, EMBED)
  hidx_all = jnp.concatenate([hxc, hxq], axis=1)
  hbit_all = jnp.concatenate([bxc, bxq], axis=1)
  cpk = jnp.concatenate([color_table[:ch], color_table[ch:]], axis=1)
  qpk = jnp.concatenate([question_table[:qh], question_table[qh:]], axis=1)
  return _sc_kernel(hbit_all, hidx_all, cpk, qpk)


# speedup vs baseline: 3.2534x; 3.1993x over previous
import jax
import jax.numpy as jnp
from jax import lax
from jax.experimental import pallas as pl
from jax.experimental.pallas import tpu as pltpu
from jax.experimental.pallas import tpu_sc as plsc

NC = 2
NS = 16
NW = NC * NS

BATCH = 16384
EMBED = 64
PITCH = 2 * EMBED
CROWS = 1000
QROWS = 1000000
BPW = BATCH // NW
BLK = 128
NBLK = BPW // BLK
LANES = 16
KV = EMBED // LANES


def _make_kernel():
  mesh = plsc.VectorSubcoreMesh(core_axis_name="c", subcore_axis_name="s")

  @pl.kernel(
      out_type=jax.ShapeDtypeStruct((BATCH, PITCH), jnp.float32),
      mesh=mesh,
      scratch_types=[
          pltpu.VMEM((2, BPW), jnp.int32),
          pltpu.VMEM((NBLK, BLK), jnp.int32),
          pltpu.VMEM((2, BLK, PITCH), jnp.float32),
          pltpu.VMEM((BPW, PITCH), jnp.float32),
          [pltpu.SemaphoreType.DMA] * NBLK,
          pltpu.SemaphoreType.DMA,
          pltpu.SemaphoreType.DMA,
      ],
  )
  def k(idx_hbm, cidx_hbm, cpk_hbm, qtab_hbm, out_hbm,
        idx_s, cidx_v, bufc, mix, qsems, csem, osem):
    wid = lax.axis_index("s") * NC + lax.axis_index("c")
    base = wid * BPW
    qtab3 = qtab_hbm

    pltpu.sync_copy(idx_hbm.at[wid], idx_s)
    pltpu.sync_copy(cidx_hbm.at[wid], cidx_v)

    def issue_q(b):
      def vec_group(v, _):
        r0 = b * BLK + v * LANES
        qvec = idx_s[1, pl.ds(r0, LANES)]
        for j in range(LANES):
          q = qvec[j]
          pltpu.async_copy(
              qtab3.at[q >> 3, q & 7],
              mix.at[r0 + j, pl.ds(EMBED, EMBED)],
              qsems[b],
          )
        return 0

      lax.fori_loop(0, BLK // LANES, vec_group, 0)

    for b in range(NBLK):
      issue_q(b)

    def stream_color(b):
      pltpu.async_copy(cpk_hbm.at[cidx_v.at[b]], bufc.at[b % 2], csem)

    def assemble_color(b):
      pltpu.make_async_copy(cpk_hbm.at[cidx_v.at[0]], bufc.at[b % 2], csem).wait()

      def vec_group(v, _):
        r0 = b * BLK + v * LANES
        cvec = idx_s[0, pl.ds(r0, LANES)]
        for l in range(LANES):
          ch = (cvec[l] & 1) * EMBED
          for t in range(KV):
            mix[r0 + l, pl.ds(t * LANES, LANES)] = bufc[
                b % 2, v * LANES + l, pl.ds(ch + t * LANES, LANES)
            ]
        return 0

      lax.fori_loop(0, BLK // LANES, vec_group, 0)

    def drain_q(b):
      pltpu.make_async_copy(
          out_hbm.at[pl.ds(0, BLK // 2)],
          mix.at[pl.ds(0, BLK // 2)],
          qsems[b],
      ).wait()

    stream_color(0)
    for b in range(NBLK):
      if b + 1 < NBLK:
        stream_color(b + 1)
      assemble_color(b)
      drain_q(b)
      pltpu.async_copy(
          mix.at[pl.ds(b * BLK, BLK)],
          out_hbm.at[pl.ds(base + b * BLK, BLK)],
          osem,
      )
    for _ in range(NBLK):
      pltpu.make_async_copy(
          mix.at[pl.ds(0, BLK)], out_hbm.at[pl.ds(base, BLK)], osem
      ).wait()

  return k


_kernel = _make_kernel()


@jax.jit
def kernel(x, color_table, question_table):
  xi = x.astype(jnp.int32).T.reshape(2, NW, BPW).transpose(1, 0, 2)
  cidx = (x[:, 0].astype(jnp.int32) >> 1).reshape(NW, NBLK, BLK)
  cpk = color_table.reshape(CROWS // 2, PITCH)
  qtab3 = question_table.reshape(QROWS // 8, 8, EMBED)
  return _kernel(xi, cidx, cpk, qtab3)
